# baseline (speedup 1.0000x reference)
import jax
import jax.numpy as jnp
from jax import lax
from jax.experimental import pallas as pl
from jax.experimental.pallas import tpu as pltpu
from jax.experimental.pallas import tpu_sc as plsc

_NC = 2
_NS = 16
_NW = _NC * _NS

_SC_ROWS = 1024
_CH = 32
_NSLOT = 1

_TC_NCH = 4


def _sc_copy_body(tab_hbm, out_hbm, buf, in_sems, out_sems):
    rows_per_w = _SC_ROWS // _NW
    nchunk = rows_per_w // _CH
    wid = lax.axis_index("s") * _NC + lax.axis_index("c")
    base = wid * rows_per_w
    for c in range(nchunk):
        slot = c % _NSLOT
        cp_in = pltpu.make_async_copy(
            tab_hbm.at[pl.ds(base + c * _CH, _CH)], buf.at[slot, :, 0],
            in_sems.at[slot])
        cp_in.start()
        cp_in.wait()
        cp_out = pltpu.make_async_copy(
            buf.at[slot], out_hbm.at[pl.ds(base + c * _CH, _CH)],
            out_sems.at[slot])
        cp_out.start()
        cp_out.wait()


def _tc_copy_body(tab_ref, out_ref, buf, in_sems, out_sems):
    s = out_ref.shape[0]
    ch = s // _TC_NCH
    ins, outs = [], []
    for i in range(_TC_NCH):
        c = pltpu.make_async_copy(
            tab_ref.at[pl.ds(_SC_ROWS + i * ch, ch)], buf.at[i, :, 0],
            in_sems.at[i])
        c.start()
        ins.append(c)
    for i in range(_TC_NCH):
        ins[i].wait()
        c = pltpu.make_async_copy(
            buf.at[i], out_ref.at[pl.ds(i * ch, ch)], out_sems.at[i])
        c.start()
        outs.append(c)
    for c in outs:
        c.wait()


def kernel(x, pos_table):
    s = x.shape[0]
    n, e = pos_table.shape
    mesh = plsc.VectorSubcoreMesh(core_axis_name="c", subcore_axis_name="s")
    sc_k = pl.kernel(
        _sc_copy_body,
        out_type=jax.ShapeDtypeStruct((_SC_ROWS, 1, e), pos_table.dtype),
        mesh=mesh,
        scratch_types=[
            pltpu.VMEM((_NSLOT, _CH, 1, e), pos_table.dtype),
            pltpu.SemaphoreType.DMA((_NSLOT,)),
            pltpu.SemaphoreType.DMA((_NSLOT,)),
        ],
    )
    sc_out = sc_k(pos_table)

    tc_rows = s - _SC_ROWS
    tc_out = pl.pallas_call(
        _tc_copy_body,
        in_specs=[pl.BlockSpec(memory_space=pl.ANY)],
        out_specs=pl.BlockSpec(memory_space=pl.ANY),
        out_shape=jax.ShapeDtypeStruct((tc_rows, 1, e), pos_table.dtype),
        scratch_shapes=[
            pltpu.VMEM((_TC_NCH, tc_rows // _TC_NCH, 1, e), pos_table.dtype),
            pltpu.SemaphoreType.DMA((_TC_NCH,)),
            pltpu.SemaphoreType.DMA((_TC_NCH,)),
        ],
    )(pos_table)
    return jnp.concatenate([sc_out, tc_out], axis=0)


# speedup vs baseline: 3.1134x; 3.1134x over previous
import functools

import jax
import jax.numpy as jnp
from jax import lax
from jax.experimental import pallas as pl
from jax.experimental.pallas import tpu as pltpu
from jax.experimental.pallas import tpu_sc as plsc

_NC = 2
_NS = 16
_NW = _NC * _NS

_CH = 16
_NSLOT = 7


def _sc_copy_body(s, e, tab_hbm, out_hbm, buf, in_sems, out_sems):
    rows_per_w = s // _NW
    nchunk = rows_per_w // _CH
    wid = lax.axis_index("s") * _NC + lax.axis_index("c")
    base = wid * rows_per_w

    def start_in(c, slot):
        cp = pltpu.make_async_copy(
            tab_hbm.at[pl.ds(base + c * _CH, _CH)], buf.at[slot, :, 0],
            in_sems.at[slot])
        cp.start()
        return cp

    def start_out(c, slot):
        cp = pltpu.make_async_copy(
            buf.at[slot], out_hbm.at[pl.ds(base + c * _CH, _CH)],
            out_sems.at[slot])
        cp.start()
        return cp

    ins = [None] * nchunk
    outs = [None] * nchunk
    for c in range(min(_NSLOT, nchunk)):
        ins[c] = start_in(c, c)
    for c in range(nchunk):
        slot = c % _NSLOT
        ins[c].wait()
        outs[c] = start_out(c, slot)
        nxt = c + _NSLOT
        if nxt < nchunk:
            outs[c].wait()
            ins[nxt] = start_in(nxt, slot)
    for c in range(max(nchunk - _NSLOT, 0), nchunk):
        outs[c].wait()


def kernel(x, pos_table):
    s = x.shape[0]
    n, e = pos_table.shape
    mesh = plsc.VectorSubcoreMesh(core_axis_name="c", subcore_axis_name="s")
    k = pl.kernel(
        functools.partial(_sc_copy_body, s, e),
        out_type=jax.ShapeDtypeStruct((s, 1, e), pos_table.dtype),
        mesh=mesh,
        scratch_types=[
            pltpu.VMEM((_NSLOT, _CH, 1, e), pos_table.dtype),
            pltpu.SemaphoreType.DMA((_NSLOT,)),
            pltpu.SemaphoreType.DMA((_NSLOT,)),
        ],
    )
    return k(pos_table)
